# baseline (speedup 1.0000x reference)
import functools

import jax
import jax.numpy as jnp
from jax import lax
from jax.experimental import pallas as pl
from jax.experimental.pallas import tpu as pltpu
from jax.experimental.pallas import tpu_sc as plsc

_W = 32
_L = 16
_BIG = 2**30


def _sc_body(rows, n_cols, shard, logits_ref, out_ref, slab_a, slab_b, outbuf_ref, sem_a, sem_b):
    wid = lax.axis_index("s") * 2 + lax.axis_index("c")
    base = wid * shard
    nv = shard // _L
    minf = jnp.full((_L,), -jnp.inf, jnp.float32)
    zero = jnp.zeros((_L,), jnp.float32)
    zeroi = jnp.zeros((_L,), jnp.int32)

    slabs = (slab_a, slab_b)
    sems = (sem_a, sem_b)

    def copy(r):
        b = r % 2
        return pltpu.make_async_copy(
            logits_ref.at[pl.ds(r * n_cols + base, shard)], slabs[b], sems[b]
        )

    copy(0).start()
    for r in range(rows):
        if r + 1 < rows:
            copy(r + 1).start()
        copy(r).wait()
        slab_ref = slabs[r % 2]

        def p1(j, carry):
            m, w = carry
            v = slab_ref[pl.ds(j * _L, _L)]
            w = jnp.where(v > m, j, w)
            m = jnp.maximum(m, v)
            return m, w

        m16, w16 = lax.fori_loop(0, nv, p1, (minf, zeroi), unroll=4)

        def p2(j, carry):
            s, t = carry
            v = slab_ref[pl.ds(j * _L, _L)]
            e = jnp.exp(v - m16)
            return s + e, t + v * e

        s16, t16 = lax.fori_loop(0, nv, p2, (zero, zero), unroll=4)
        outbuf_ref[r, 0, :] = m16
        outbuf_ref[r, 1, :] = s16
        outbuf_ref[r, 2, :] = t16
        outbuf_ref[r, 3, :] = w16.astype(jnp.float32)

    pltpu.sync_copy(outbuf_ref, out_ref.at[wid])


def _merge_body(n_cols, covered, shard, p_ref, tail_ref, samp_ref, ent_ref):
    p = p_ref[...]
    m_w = p[:, :, 0, :]
    s_w = p[:, :, 1, :]
    t_w = p[:, :, 2, :]
    w_w = p[:, :, 3, :].astype(jnp.int32)

    tail = tail_ref[...]
    col = covered + jax.lax.broadcasted_iota(jnp.int32, tail.shape, 1)
    valid = col < n_cols
    xt = jnp.where(valid, tail, -jnp.inf)

    big_m = jnp.maximum(jnp.max(m_w, axis=(0, 2)), jnp.max(xt, axis=1))
    a_w = jnp.exp(m_w - big_m[None, :, None])
    et = jnp.exp(xt - big_m[:, None])
    s = jnp.sum(s_w * a_w, axis=(0, 2)) + jnp.sum(et, axis=1)
    t = jnp.sum(t_w * a_w, axis=(0, 2)) + jnp.sum(jnp.where(valid, xt * et, 0.0), axis=1)
    ent_ref[...] = ((big_m + jnp.log(s)) - t / s).reshape(-1, 1)

    base_w = jax.lax.broadcasted_iota(jnp.int32, w_w.shape, 0) * shard
    lane = jax.lax.broadcasted_iota(jnp.int32, w_w.shape, 2)
    idx = base_w + w_w * _L + lane
    cand_w = jnp.min(
        jnp.where(m_w == big_m[None, :, None], idx, _BIG), axis=(0, 2)
    )
    cand_t = jnp.min(jnp.where(xt == big_m[:, None], col, _BIG), axis=1)
    samp_ref[...] = jnp.minimum(cand_w, cand_t).reshape(-1, 1)


def kernel(logits):
    rows, n_cols = logits.shape
    shard = (n_cols // _W) // _L * _L
    covered = _W * shard

    mesh = plsc.VectorSubcoreMesh(
        core_axis_name="c", subcore_axis_name="s", num_cores=2, num_subcores=16
    )
    partials = pl.kernel(
        functools.partial(_sc_body, rows, n_cols, shard),
        out_type=jax.ShapeDtypeStruct((_W, rows, 4, _L), jnp.float32),
        mesh=mesh,
        scratch_types=[
            pltpu.VMEM((shard,), jnp.float32),
            pltpu.VMEM((shard,), jnp.float32),
            pltpu.VMEM((rows, 4, _L), jnp.float32),
            pltpu.SemaphoreType.DMA,
            pltpu.SemaphoreType.DMA,
        ],
    )(logits.reshape(-1))

    samp, ent = pl.pallas_call(
        functools.partial(_merge_body, n_cols, covered, shard),
        grid=(1,),
        in_specs=[
            pl.BlockSpec((_W, rows, 4, _L), lambda i: (0, 0, 0, 0)),
            pl.BlockSpec((rows, 128), lambda i: (0, covered // 128)),
        ],
        out_specs=[
            pl.BlockSpec((rows, 1), lambda i: (0, 0)),
            pl.BlockSpec((rows, 1), lambda i: (0, 0)),
        ],
        out_shape=[
            jax.ShapeDtypeStruct((rows, 1), jnp.int32),
            jax.ShapeDtypeStruct((rows, 1), jnp.float32),
        ],
    )(partials, logits)
    return (samp.reshape(rows), logits, ent.reshape(rows))


# speedup vs baseline: 20.2529x; 20.2529x over previous
import functools

import jax
import jax.numpy as jnp
from jax.experimental import pallas as pl
from jax.experimental.pallas import tpu as pltpu

_CHUNK = 65536
_LANES = 128
_BIG = 2**30


def _maxpass(x_ref, rows, jpg, base_j, m0, w0, masked, n_cols, chunk_base):
    def body(j, carry):
        m, w = carry
        x = x_ref[:, pl.ds(j * _LANES, _LANES)]
        if masked:
            col = chunk_base + j * _LANES + jax.lax.broadcasted_iota(
                jnp.int32, (rows, _LANES), 1
            )
            x = jnp.where(col < n_cols, x, -jnp.inf)
        imp = x > m
        w = jnp.where(imp, base_j + j, w)
        m = jnp.maximum(m, x)
        return m, w

    return jax.lax.fori_loop(0, jpg, body, (m0, w0), unroll=64)


def _sumpass(x_ref, rows, jpg, m, masked, n_cols, chunk_base):
    zero = jnp.zeros((rows, _LANES), jnp.float32)

    def body(j, carry):
        s, t = carry
        x = x_ref[:, pl.ds(j * _LANES, _LANES)]
        if masked:
            col = chunk_base + j * _LANES + jax.lax.broadcasted_iota(
                jnp.int32, (rows, _LANES), 1
            )
            x = jnp.where(col < n_cols, x, -jnp.inf)
        e = jnp.exp(x - m)
        xe = x * e
        if masked:
            xe = jnp.where(col < n_cols, xe, 0.0)
        return s + e, t + xe

    return jax.lax.fori_loop(0, jpg, body, (zero, zero), unroll=64)


def _chunk_update(x_ref, i, rows, jpg, n_cols, masked, m_ref, s_ref, t_ref, w_ref):
    chunk_base = i * _CHUNK
    m_old = m_ref[...]
    m_new, w_new = _maxpass(
        x_ref, rows, jpg, i * jpg, m_old, w_ref[...], masked, n_cols, chunk_base
    )
    w_ref[...] = w_new
    m_ref[...] = m_new
    s_c, t_c = _sumpass(x_ref, rows, jpg, m_new, masked, n_cols, chunk_base)
    alpha = jnp.exp(m_old - m_new)
    s_ref[...] = s_ref[...] * alpha + s_c
    t_ref[...] = t_ref[...] * alpha + t_c


def _body(n_cols, n_chunks, x_ref, samp_ref, ent_ref, m_ref, s_ref, t_ref, w_ref):
    i = pl.program_id(0)
    rows = x_ref.shape[0]
    jpg = _CHUNK // _LANES

    @pl.when(i == 0)
    def _init():
        m_ref[...] = jnp.full((rows, _LANES), -jnp.inf, jnp.float32)
        s_ref[...] = jnp.zeros((rows, _LANES), jnp.float32)
        t_ref[...] = jnp.zeros((rows, _LANES), jnp.float32)
        w_ref[...] = jnp.zeros((rows, _LANES), jnp.int32)

    @pl.when(i < n_chunks - 1)
    def _main():
        _chunk_update(x_ref, i, rows, jpg, n_cols, False, m_ref, s_ref, t_ref, w_ref)

    @pl.when(i == n_chunks - 1)
    def _last():
        _chunk_update(x_ref, i, rows, jpg, n_cols, True, m_ref, s_ref, t_ref, w_ref)

        m_lane = m_ref[...]
        big_m = jnp.max(m_lane, axis=1, keepdims=True)
        a_f = jnp.exp(m_lane - big_m)
        s = jnp.sum(s_ref[...] * a_f, axis=1, keepdims=True)
        t = jnp.sum(t_ref[...] * a_f, axis=1, keepdims=True)
        ent_ref[...] = (big_m + jnp.log(s)) - t / s
        lane = jax.lax.broadcasted_iota(jnp.int32, (rows, _LANES), 1)
        idx = w_ref[...] * _LANES + lane
        cand = jnp.where(m_lane == big_m, idx, _BIG)
        samp_ref[...] = jnp.min(cand, axis=1, keepdims=True)


def kernel(logits):
    rows, n_cols = logits.shape
    n_chunks = pl.cdiv(n_cols, _CHUNK)
    samp, ent = pl.pallas_call(
        functools.partial(_body, n_cols, n_chunks),
        grid=(n_chunks,),
        in_specs=[pl.BlockSpec((rows, _CHUNK), lambda i: (0, i))],
        out_specs=[
            pl.BlockSpec((rows, 1), lambda i: (0, 0)),
            pl.BlockSpec((rows, 1), lambda i: (0, 0)),
        ],
        out_shape=[
            jax.ShapeDtypeStruct((rows, 1), jnp.int32),
            jax.ShapeDtypeStruct((rows, 1), jnp.float32),
        ],
        scratch_shapes=[
            pltpu.VMEM((rows, _LANES), jnp.float32),
            pltpu.VMEM((rows, _LANES), jnp.float32),
            pltpu.VMEM((rows, _LANES), jnp.float32),
            pltpu.VMEM((rows, _LANES), jnp.int32),
        ],
    )(logits)
    return (samp.reshape(rows), logits, ent.reshape(rows))
